# Initial kernel scaffold; baseline (speedup 1.0000x reference)
import jax
import jax.numpy as jnp
from jax.experimental import pallas as pl


def kernel(x, W):
    raise NotImplementedError("write your pallas kernel here")


# speedup vs baseline: 1.9621x; 1.9621x over previous
import functools

import jax
import jax.numpy as jnp
from jax import lax
from jax.experimental import pallas as pl
from jax.experimental.pallas import tpu as pltpu
from jax.experimental.pallas import tpu_sc as plsc

_BT = 16 * 576
_D = 256
_S = 4
_K = 1024
_SUB = _D // _S
_BLK = 512

_NC, _NS = 2, 16
_NW = _NC * _NS
_TCH = 8
_BPW = _BT // _TCH
_IDXC = 128
_NSTRM = _BPW // _IDXC


def _dist_body(x_ref, wt_ref, idx_ref, err_ref):
    i = pl.program_id(0)

    @pl.when(i == 0)
    def _init():
        err_ref[0, 0] = 0.0

    xb = x_ref[...]
    acc = jnp.float32(0.0)
    for s in range(_S):
        xi = xb[:, s * _SUB:(s + 1) * _SUB]
        wt = wt_ref[s]
        xnorm = jnp.sum(xi * xi, axis=1, keepdims=True)
        cbnorm = jnp.sum(wt * wt, axis=0, keepdims=True)
        scores = jnp.dot(xi, wt, preferred_element_type=jnp.float32)
        d = (xnorm + cbnorm) - 2.0 * scores
        m = jnp.min(d, axis=1, keepdims=True)
        iota = lax.broadcasted_iota(jnp.int32, d.shape, 1)
        sel = jnp.where(d == m, iota, jnp.int32(2 ** 30))
        idx = jnp.min(sel, axis=1).astype(jnp.int32)
        idx_ref[s, :] = idx + s * _K
        acc = acc + jnp.sum(m)
    err_ref[0, 0] += acc * (1.25 / (_BT * _SUB))


def _distances(xf, wt):
    return pl.pallas_call(
        _dist_body,
        grid=(_BT // _BLK,),
        in_specs=[
            pl.BlockSpec((_BLK, _D), lambda i: (i, 0)),
            pl.BlockSpec((_S, _SUB, _K), lambda i: (0, 0, 0)),
        ],
        out_specs=[
            pl.BlockSpec((_S, _BLK), lambda i: (0, i)),
            pl.BlockSpec(memory_space=pltpu.SMEM),
        ],
        out_shape=[
            jax.ShapeDtypeStruct((_S, _BT), jnp.int32),
            jax.ShapeDtypeStruct((1, 1), jnp.float32),
        ],
    )(xf, wt)


def _gather_body(table_ref, idx_ref, out_ref, idx_v, rows_v, sem):
    wid = lax.axis_index("s") * _NC + lax.axis_index("c")
    split = wid // _TCH
    tchunk = lax.rem(wid, _TCH)
    base = pl.multiple_of(wid * _BPW, _IDXC)
    pltpu.sync_copy(idx_ref.at[pl.ds(base, _BPW)], idx_v)
    copies = []
    for j in range(_NSTRM):
        copies.append(pltpu.async_copy(
            table_ref.at[idx_v.at[pl.ds(j * _IDXC, _IDXC)]],
            rows_v.at[pl.ds(j * _IDXC, _IDXC)],
            sem,
        ))
    for cp in copies:
        cp.wait()
    tbase = pl.multiple_of(tchunk * _BPW, _IDXC)
    pltpu.sync_copy(rows_v, out_ref.at[split, pl.ds(tbase, _BPW)])


@functools.lru_cache(maxsize=1)
def _gather_kernel():
    return pl.kernel(
        _gather_body,
        out_type=jax.ShapeDtypeStruct((_S, _BT, _SUB), jnp.float32),
        mesh=plsc.VectorSubcoreMesh(
            core_axis_name="c", subcore_axis_name="s",
            num_cores=_NC, num_subcores=_NS),
        scratch_types=[
            pltpu.VMEM((_BPW,), jnp.int32),
            pltpu.VMEM((_BPW, _SUB), jnp.float32),
            pltpu.SemaphoreType.DMA,
        ],
        compiler_params=pltpu.CompilerParams(use_tc_tiling_on_sc=False),
    )


def kernel(x, W):
    B, T, D = x.shape
    xf = x.reshape(B * T, D)
    wt = W.transpose(0, 2, 1)
    idxg, err = _distances(xf, wt)
    table = W.reshape(_S * _K, _SUB)
    idx_flat = idxg.reshape(_S * _BT)
    quant = _gather_kernel()(table, idx_flat)
    quant = quant.transpose(1, 0, 2).reshape(B, T, D)
    return quant, err[0, 0]
